# baseline (speedup 1.0000x reference)
import functools

import jax
import jax.numpy as jnp
from jax import lax
from jax.experimental import pallas as pl
from jax.experimental.pallas import tpu as pltpu
from jax.experimental.pallas import tpu_sc as plsc

N_NODES = 10000
N_EDGES = 320000
NP = 10240
NC, NS = 2, 16
NW = NC * NS
EP = 327680
PAD_DST = 10100
ROWS_A = 80
ROWS_C = 160
CHUNK_ROWS = 160
N_CHUNKS = NP // CHUNK_ROWS // NC
TILE_ROWS = CHUNK_ROWS // NS
CHUNK_WORDS = CHUNK_ROWS * NP
TILE_WORDS = TILE_ROWS * NP
BR = 8
NB = ROWS_C // (2 * BR)
NEG_SLOPE = 0.01

_mesh = plsc.VectorSubcoreMesh(
    core_axis_name="c", subcore_axis_name="s", num_cores=NC, num_subcores=NS)


@functools.partial(
    pl.kernel,
    out_type=jax.ShapeDtypeStruct((NC, NP), jnp.float32),
    mesh=_mesh,
    scratch_types=[
        pltpu.VMEM((ROWS_A * 128,), jnp.int32),
        pltpu.VMEM((ROWS_A * 128,), jnp.float32),
        pltpu.VMEM_SHARED((NP,), jnp.float32),
    ],
)
def _deg_kernel(dst_hbm, zeros_hbm, out_hbm, idx_v, ones_v, deg_sh):
    c = lax.axis_index("c")
    s = lax.axis_index("s")
    w = s * NC + c
    pltpu.sync_copy(dst_hbm.at[w], idx_v)

    def ones_body(i, _):
        ones_v[pl.ds(i * 16, 16)] = jnp.full((16,), 1.0, jnp.float32)
        return 0

    lax.fori_loop(0, ROWS_A * 8, ones_body, 0)

    @pl.when(s == 0)
    def _():
        pltpu.sync_copy(zeros_hbm.at[pl.ds(0, NP)], deg_sh)

    plsc.subcore_barrier()
    pltpu.sync_copy(ones_v, deg_sh.at[idx_v], add=True)
    plsc.subcore_barrier()

    @pl.when(s == 0)
    def _():
        pltpu.sync_copy(deg_sh, out_hbm.at[c])


def _dinv_body(part_ref, out_ref):
    p = part_ref[...]
    deg = p[0:80, :] + p[80:160, :] + 1.0
    dinv = lax.rsqrt(deg)
    gi = (lax.broadcasted_iota(jnp.int32, (80, 128), 0) * 128
          + lax.broadcasted_iota(jnp.int32, (80, 128), 1))
    out_ref[...] = jnp.where(gi < N_NODES, dinv, 0.0)


def _dinv(partials):
    return pl.pallas_call(
        _dinv_body,
        out_shape=jax.ShapeDtypeStruct((80, 128), jnp.float32),
    )(partials.reshape(160, 128)).reshape(NP)


@functools.partial(
    pl.kernel,
    out_type=jax.ShapeDtypeStruct((NP * NP,), jnp.float32),
    mesh=_mesh,
    scratch_types=[
        pltpu.VMEM((ROWS_C, 128), jnp.int32),
        pltpu.VMEM((BR * 128,), jnp.int32),
        pltpu.VMEM((BR * 128,), jnp.int32),
        pltpu.VMEM((BR * 128,), jnp.float32),
        pltpu.VMEM((BR * 128,), jnp.float32),
        pltpu.VMEM((128,), jnp.int32),
        pltpu.VMEM((128,), jnp.float32),
        pltpu.SemaphoreType.DMA,
        pltpu.SemaphoreType.DMA,
        pltpu.VMEM_SHARED((CHUNK_WORDS,), jnp.float32),
    ],
)
def _abuild_kernel(dst_hbm, src_hbm, zeros_hbm, a_hbm,
                   flat_v, ix0_v, ix1_v, wv0_v, wv1_v, dgi_v, dgv_v,
                   ssem, zsem, chunk_sh):
    c = lax.axis_index("c")
    s = lax.axis_index("s")

    def init_body(t, _):
        pltpu.sync_copy(dst_hbm.at[s, pl.ds(t * BR * 128, BR * 128)], ix0_v)
        pltpu.sync_copy(src_hbm.at[s, pl.ds(t * BR * 128, BR * 128)], ix1_v)
        for r in range(BR):
            for g in range(8):
                sl = pl.ds(g * 16, 16)
                fsl = pl.ds(r * 128 + g * 16, 16)
                flat_v[t * BR + r, sl] = ix0_v[fsl] * NP + ix1_v[fsl]
        return 0

    lax.fori_loop(0, ROWS_C // BR, init_body, 0)

    lanes = lax.iota(jnp.int32, 16)
    my_zero = s * TILE_WORDS

    pltpu.async_copy(zeros_hbm, chunk_sh.at[pl.ds(my_zero, TILE_WORDS)], zsem)

    def fill(j, dst_ix, dst_wv, c0):
        for r in range(BR):
            row = j * BR + r
            for g in range(8):
                sl = pl.ds(g * 16, 16)
                fsl = pl.ds(r * 128 + g * 16, 16)
                rr = flat_v[row, sl] - c0
                m = (rr >= 0) & (rr < CHUNK_WORDS)
                dst_ix[fsl] = jnp.where(m, rr, 0)
                dst_wv[fsl] = jnp.where(m, 1.0, 0.0)

    def chunk_body(cb, _):
        base = (cb * NC + c) * CHUNK_ROWS
        c0 = base * NP

        pltpu.make_async_copy(
            zeros_hbm, chunk_sh.at[pl.ds(my_zero, TILE_WORDS)], zsem).wait()
        plsc.subcore_barrier()

        def sbatch(it, _):
            @pl.when(it >= 1)
            def _():
                pltpu.make_async_copy(
                    wv0_v, chunk_sh.at[ix0_v], ssem).wait()

            fill(it * 2, ix0_v, wv0_v, c0)
            pltpu.async_copy(wv0_v, chunk_sh.at[ix0_v], ssem, add=True)

            @pl.when(it >= 1)
            def _():
                pltpu.make_async_copy(
                    wv1_v, chunk_sh.at[ix1_v], ssem).wait()

            fill(it * 2 + 1, ix1_v, wv1_v, c0)
            pltpu.async_copy(wv1_v, chunk_sh.at[ix1_v], ssem, add=True)
            return 0

        lax.fori_loop(0, NB, sbatch, 0)

        for g in range(1, 8):
            sl = pl.ds(g * 16, 16)
            dgi_v[sl] = jnp.zeros((16,), jnp.int32)
            dgv_v[sl] = jnp.zeros((16,), jnp.float32)
        l = s * TILE_ROWS + lanes
        dm = lanes < TILE_ROWS
        dgi_v[pl.ds(0, 16)] = jnp.where(dm, l * NP + base + l, 0)
        dgv_v[pl.ds(0, 16)] = jnp.where(dm, 1.0, 0.0)
        pltpu.async_copy(dgv_v, chunk_sh.at[dgi_v], ssem, add=True)

        pltpu.make_async_copy(wv0_v, chunk_sh.at[ix0_v], ssem).wait()
        pltpu.make_async_copy(wv1_v, chunk_sh.at[ix1_v], ssem).wait()
        pltpu.make_async_copy(dgv_v, chunk_sh.at[dgi_v], ssem).wait()
        plsc.subcore_barrier()

        pltpu.sync_copy(
            chunk_sh.at[pl.ds(my_zero, TILE_WORDS)],
            a_hbm.at[pl.ds(base * NP + my_zero, TILE_WORDS)])
        pltpu.async_copy(
            zeros_hbm, chunk_sh.at[pl.ds(my_zero, TILE_WORDS)], zsem)
        return 0

    lax.fori_loop(0, N_CHUNKS, chunk_body, 0)
    pltpu.make_async_copy(
        zeros_hbm, chunk_sh.at[pl.ds(my_zero, TILE_WORDS)], zsem).wait()


def _mm_body(h_ref, w_ref, d_ref, o_ref):
    o_ref[...] = jnp.dot(h_ref[...], w_ref[...],
                         preferred_element_type=jnp.float32) * d_ref[:, 0:1]


def _mm(h, w, dinv_bc):
    m, din = h.shape
    dout = w.shape[1]
    return pl.pallas_call(
        _mm_body,
        grid=(m // 256,),
        in_specs=[
            pl.BlockSpec((256, din), lambda i: (i, 0)),
            pl.BlockSpec((din, dout), lambda i: (0, 0)),
            pl.BlockSpec((256, 128), lambda i: (i, 0)),
        ],
        out_specs=pl.BlockSpec((256, dout), lambda i: (i, 0)),
        out_shape=jax.ShapeDtypeStruct((m, dout), jnp.float32),
    )(h, w, dinv_bc)


def _smm_body(nk, s_ref, h_ref, b_ref, d_ref, o_ref, acc_ref):
    k = pl.program_id(1)

    @pl.when(k == 0)
    def _():
        acc_ref[...] = jnp.zeros_like(acc_ref)

    acc_ref[...] += jnp.dot(s_ref[...], h_ref[...],
                            preferred_element_type=jnp.float32)

    @pl.when(k == nk - 1)
    def _():
        t = acc_ref[...] * d_ref[:, 0:1] + b_ref[...]
        o_ref[...] = jnp.where(t > 0, t, NEG_SLOPE * t)


def _smm(S, h, b, dinv_bc):
    dout = h.shape[1]
    nk = NP // 512
    return pl.pallas_call(
        functools.partial(_smm_body, nk),
        grid=(NP // 256, nk),
        in_specs=[
            pl.BlockSpec((256, 512), lambda i, k: (i, k)),
            pl.BlockSpec((512, dout), lambda i, k: (k, 0)),
            pl.BlockSpec((1, dout), lambda i, k: (0, 0)),
            pl.BlockSpec((256, 128), lambda i, k: (i, 0)),
        ],
        out_specs=pl.BlockSpec((256, dout), lambda i, k: (i, 0)),
        out_shape=jax.ShapeDtypeStruct((NP, dout), jnp.float32),
        scratch_shapes=[pltpu.VMEM((256, dout), jnp.float32)],
        compiler_params=pltpu.CompilerParams(
            dimension_semantics=("parallel", "arbitrary")),
    )(S, h, b.reshape(1, dout), dinv_bc)


def _final_body(nm, h_ref, wf_ref, bf_ref, o_ref, acc_ref):
    i = pl.program_id(0)

    @pl.when(i == 0)
    def _():
        acc_ref[...] = jnp.zeros_like(acc_ref)

    row = i * 256 + lax.broadcasted_iota(jnp.int32, (256, 256), 0)
    x = jnp.where(row < N_NODES, h_ref[...], 0.0)
    acc_ref[...] += jnp.sum(x, axis=0, keepdims=True)

    @pl.when(i == nm - 1)
    def _():
        o_ref[...] = (jnp.dot(acc_ref[...], wf_ref[...],
                              preferred_element_type=jnp.float32)
                      + float(N_NODES) * bf_ref[...])


def _final(h, wf, bf):
    nm = NP // 256
    return pl.pallas_call(
        functools.partial(_final_body, nm),
        grid=(nm,),
        in_specs=[
            pl.BlockSpec((256, 256), lambda i: (i, 0)),
            pl.BlockSpec((256, 1024), lambda i: (0, 0)),
            pl.BlockSpec((1, 1024), lambda i: (0, 0)),
        ],
        out_specs=pl.BlockSpec((1, 1024), lambda i: (0, 0)),
        out_shape=jax.ShapeDtypeStruct((1, 1024), jnp.float32),
        scratch_shapes=[pltpu.VMEM((1, 256), jnp.float32)],
    )(h, wf, bf.reshape(1, 1024))


def kernel(x, edge_index, W1, b1, W2, b2, W3, b3, W4, b4, W5, b5, W6, b6,
           Wf, bf):
    src = edge_index[0]
    dst = edge_index[1]
    npad = EP - N_EDGES
    dstp = jnp.concatenate(
        [dst, jnp.full((npad,), PAD_DST, jnp.int32)])
    srcp = jnp.concatenate([src, jnp.zeros((npad,), jnp.int32)])
    zeros_tile = jnp.zeros((TILE_WORDS,), jnp.float32)

    partials = _deg_kernel(dstp.reshape(NW, ROWS_A * 128), zeros_tile)
    dinv = _dinv(partials)
    a_flat = _abuild_kernel(dstp.reshape(NS, ROWS_C * 128),
                            srcp.reshape(NS, ROWS_C * 128), zeros_tile)
    A = a_flat.reshape(NP, NP)
    dinv_bc = jnp.broadcast_to(dinv[:, None], (NP, 128))

    xp = jnp.pad(x, ((0, NP - N_NODES), (0, 0)))
    h = xp
    for W, b in ((W1, b1), (W2, b2), (W3, b3), (W4, b4), (W5, b5), (W6, b6)):
        h = _smm(A, _mm(h, W, dinv_bc), b, dinv_bc)
    return _final(h, Wf, bf).reshape(1024)


# speedup vs baseline: 3.0276x; 3.0276x over previous
import functools

import jax
import jax.numpy as jnp
from jax import lax
from jax.experimental import pallas as pl
from jax.experimental.pallas import tpu as pltpu
from jax.experimental.pallas import tpu_sc as plsc

N_NODES = 10000
N_EDGES = 320000
NP = 10240
NC, NS = 2, 16
NW = NC * NS
EP = 327680
PAD_DST = 10100
ROWS_A = 80
ROWS_C = 160
CHUNK_ROWS = 160
N_CHUNKS = NP // CHUNK_ROWS // NC
TILE_ROWS = CHUNK_ROWS // NS
CHUNK_WORDS = CHUNK_ROWS * NP
TILE_WORDS = TILE_ROWS * NP
BR = 8
NB = ROWS_C // (2 * BR)
NEG_SLOPE = 0.01

_mesh = plsc.VectorSubcoreMesh(
    core_axis_name="c", subcore_axis_name="s", num_cores=NC, num_subcores=NS)


@functools.partial(
    pl.kernel,
    out_type=jax.ShapeDtypeStruct((NC, NP), jnp.float32),
    mesh=_mesh,
    scratch_types=[
        pltpu.VMEM((ROWS_A * 128,), jnp.int32),
        pltpu.VMEM((ROWS_A * 128,), jnp.float32),
        pltpu.VMEM_SHARED((NP,), jnp.float32),
    ],
)
def _deg_kernel(dst_hbm, zeros_hbm, out_hbm, idx_v, ones_v, deg_sh):
    c = lax.axis_index("c")
    s = lax.axis_index("s")
    w = s * NC + c
    pltpu.sync_copy(dst_hbm.at[w], idx_v)

    def ones_body(i, _):
        ones_v[pl.ds(i * 16, 16)] = jnp.full((16,), 1.0, jnp.float32)
        return 0

    lax.fori_loop(0, ROWS_A * 8, ones_body, 0)

    @pl.when(s == 0)
    def _():
        pltpu.sync_copy(zeros_hbm.at[pl.ds(0, NP)], deg_sh)

    plsc.subcore_barrier()
    pltpu.sync_copy(ones_v, deg_sh.at[idx_v], add=True)
    plsc.subcore_barrier()

    @pl.when(s == 0)
    def _():
        pltpu.sync_copy(deg_sh, out_hbm.at[c])


def _dinv_body(part_ref, out_ref):
    p = part_ref[...]
    deg = p[0:80, :] + p[80:160, :] + 1.0
    dinv = lax.rsqrt(deg)
    gi = (lax.broadcasted_iota(jnp.int32, (80, 128), 0) * 128
          + lax.broadcasted_iota(jnp.int32, (80, 128), 1))
    out_ref[...] = jnp.where(gi < N_NODES, dinv, 0.0)


def _dinv(partials):
    return pl.pallas_call(
        _dinv_body,
        out_shape=jax.ShapeDtypeStruct((80, 128), jnp.float32),
    )(partials.reshape(160, 128)).reshape(NP)


@functools.partial(
    pl.kernel,
    out_type=jax.ShapeDtypeStruct((NP * NP,), jnp.float32),
    mesh=_mesh,
    scratch_types=[
        pltpu.VMEM((ROWS_C, 128), jnp.int32),
        pltpu.VMEM((BR * 128,), jnp.int32),
        pltpu.VMEM((BR * 128,), jnp.int32),
        pltpu.VMEM((BR * 128,), jnp.float32),
        pltpu.VMEM((BR * 128,), jnp.float32),
        pltpu.VMEM((128,), jnp.int32),
        pltpu.VMEM((128,), jnp.float32),
        pltpu.SemaphoreType.DMA,
        pltpu.SemaphoreType.DMA,
        pltpu.VMEM_SHARED((CHUNK_WORDS,), jnp.float32),
    ],
)
def _abuild_kernel(dst_hbm, src_hbm, zeros_hbm, a_hbm,
                   flat_v, ix0_v, ix1_v, wv0_v, wv1_v, dgi_v, dgv_v,
                   ssem, zsem, chunk_sh):
    c = lax.axis_index("c")
    s = lax.axis_index("s")

    def init_body(t, _):
        pltpu.sync_copy(dst_hbm.at[s, pl.ds(t * BR * 128, BR * 128)], ix0_v)
        pltpu.sync_copy(src_hbm.at[s, pl.ds(t * BR * 128, BR * 128)], ix1_v)
        for r in range(BR):
            for g in range(8):
                sl = pl.ds(g * 16, 16)
                fsl = pl.ds(r * 128 + g * 16, 16)
                flat_v[t * BR + r, sl] = ix0_v[fsl] * NP + ix1_v[fsl]
        return 0

    lax.fori_loop(0, ROWS_C // BR, init_body, 0)

    lanes = lax.iota(jnp.int32, 16)
    my_zero = s * TILE_WORDS

    pltpu.async_copy(zeros_hbm, chunk_sh.at[pl.ds(my_zero, TILE_WORDS)], zsem)

    def fill(j, dst_ix, dst_wv, c0, dump0):
        for r in range(BR):
            row = j * BR + r
            for g in range(8):
                sl = pl.ds(g * 16, 16)
                fsl = pl.ds(r * 128 + g * 16, 16)
                rr = flat_v[row, sl] - c0
                m = (rr >= 0) & (rr < CHUNK_WORDS)
                dump = dump0 + (r * 128 + g * 16) + lanes
                dst_ix[fsl] = jnp.where(m, rr, dump)
                dst_wv[fsl] = jnp.where(m, 1.0, 0.0)

    def chunk_body(cb, _):
        base = (cb * NC + c) * CHUNK_ROWS
        c0 = base * NP

        pltpu.make_async_copy(
            zeros_hbm, chunk_sh.at[pl.ds(my_zero, TILE_WORDS)], zsem).wait()
        plsc.subcore_barrier()

        def sbatch(it, _):
            @pl.when(it >= 1)
            def _():
                pltpu.make_async_copy(
                    wv0_v, chunk_sh.at[ix0_v], ssem).wait()

            fill(it * 2, ix0_v, wv0_v, c0, s * 2048)
            pltpu.async_copy(wv0_v, chunk_sh.at[ix0_v], ssem, add=True)

            @pl.when(it >= 1)
            def _():
                pltpu.make_async_copy(
                    wv1_v, chunk_sh.at[ix1_v], ssem).wait()

            fill(it * 2 + 1, ix1_v, wv1_v, c0, s * 2048 + 1024)
            pltpu.async_copy(wv1_v, chunk_sh.at[ix1_v], ssem, add=True)
            return 0

        lax.fori_loop(0, NB, sbatch, 0)

        for g in range(1, 8):
            sl = pl.ds(g * 16, 16)
            dgi_v[sl] = jnp.zeros((16,), jnp.int32)
            dgv_v[sl] = jnp.zeros((16,), jnp.float32)
        l = s * TILE_ROWS + lanes
        dm = lanes < TILE_ROWS
        dgi_v[pl.ds(0, 16)] = jnp.where(dm, l * NP + base + l, 0)
        dgv_v[pl.ds(0, 16)] = jnp.where(dm, 1.0, 0.0)
        pltpu.async_copy(dgv_v, chunk_sh.at[dgi_v], ssem, add=True)

        pltpu.make_async_copy(wv0_v, chunk_sh.at[ix0_v], ssem).wait()
        pltpu.make_async_copy(wv1_v, chunk_sh.at[ix1_v], ssem).wait()
        pltpu.make_async_copy(dgv_v, chunk_sh.at[dgi_v], ssem).wait()
        plsc.subcore_barrier()

        pltpu.sync_copy(
            chunk_sh.at[pl.ds(my_zero, TILE_WORDS)],
            a_hbm.at[pl.ds(base * NP + my_zero, TILE_WORDS)])
        pltpu.async_copy(
            zeros_hbm, chunk_sh.at[pl.ds(my_zero, TILE_WORDS)], zsem)
        return 0

    lax.fori_loop(0, N_CHUNKS, chunk_body, 0)
    pltpu.make_async_copy(
        zeros_hbm, chunk_sh.at[pl.ds(my_zero, TILE_WORDS)], zsem).wait()


def _mm_body(h_ref, w_ref, d_ref, o_ref):
    o_ref[...] = jnp.dot(h_ref[...], w_ref[...],
                         preferred_element_type=jnp.float32) * d_ref[:, 0:1]


def _mm(h, w, dinv_bc):
    m, din = h.shape
    dout = w.shape[1]
    return pl.pallas_call(
        _mm_body,
        grid=(m // 256,),
        in_specs=[
            pl.BlockSpec((256, din), lambda i: (i, 0)),
            pl.BlockSpec((din, dout), lambda i: (0, 0)),
            pl.BlockSpec((256, 128), lambda i: (i, 0)),
        ],
        out_specs=pl.BlockSpec((256, dout), lambda i: (i, 0)),
        out_shape=jax.ShapeDtypeStruct((m, dout), jnp.float32),
    )(h, w, dinv_bc)


def _smm_body(nk, s_ref, h_ref, b_ref, d_ref, o_ref, acc_ref):
    k = pl.program_id(1)

    @pl.when(k == 0)
    def _():
        acc_ref[...] = jnp.zeros_like(acc_ref)

    acc_ref[...] += jnp.dot(s_ref[...], h_ref[...],
                            preferred_element_type=jnp.float32)

    @pl.when(k == nk - 1)
    def _():
        t = acc_ref[...] * d_ref[:, 0:1] + b_ref[...]
        o_ref[...] = jnp.where(t > 0, t, NEG_SLOPE * t)


def _smm(S, h, b, dinv_bc):
    dout = h.shape[1]
    nk = NP // 512
    return pl.pallas_call(
        functools.partial(_smm_body, nk),
        grid=(NP // 256, nk),
        in_specs=[
            pl.BlockSpec((256, 512), lambda i, k: (i, k)),
            pl.BlockSpec((512, dout), lambda i, k: (k, 0)),
            pl.BlockSpec((1, dout), lambda i, k: (0, 0)),
            pl.BlockSpec((256, 128), lambda i, k: (i, 0)),
        ],
        out_specs=pl.BlockSpec((256, dout), lambda i, k: (i, 0)),
        out_shape=jax.ShapeDtypeStruct((NP, dout), jnp.float32),
        scratch_shapes=[pltpu.VMEM((256, dout), jnp.float32)],
        compiler_params=pltpu.CompilerParams(
            dimension_semantics=("parallel", "arbitrary")),
    )(S, h, b.reshape(1, dout), dinv_bc)


def _final_body(nm, h_ref, wf_ref, bf_ref, o_ref, acc_ref):
    i = pl.program_id(0)

    @pl.when(i == 0)
    def _():
        acc_ref[...] = jnp.zeros_like(acc_ref)

    row = i * 256 + lax.broadcasted_iota(jnp.int32, (256, 256), 0)
    x = jnp.where(row < N_NODES, h_ref[...], 0.0)
    acc_ref[...] += jnp.sum(x, axis=0, keepdims=True)

    @pl.when(i == nm - 1)
    def _():
        o_ref[...] = (jnp.dot(acc_ref[...], wf_ref[...],
                              preferred_element_type=jnp.float32)
                      + float(N_NODES) * bf_ref[...])


def _final(h, wf, bf):
    nm = NP // 256
    return pl.pallas_call(
        functools.partial(_final_body, nm),
        grid=(nm,),
        in_specs=[
            pl.BlockSpec((256, 256), lambda i: (i, 0)),
            pl.BlockSpec((256, 1024), lambda i: (0, 0)),
            pl.BlockSpec((1, 1024), lambda i: (0, 0)),
        ],
        out_specs=pl.BlockSpec((1, 1024), lambda i: (0, 0)),
        out_shape=jax.ShapeDtypeStruct((1, 1024), jnp.float32),
        scratch_shapes=[pltpu.VMEM((1, 256), jnp.float32)],
    )(h, wf, bf.reshape(1, 1024))


def kernel(x, edge_index, W1, b1, W2, b2, W3, b3, W4, b4, W5, b5, W6, b6,
           Wf, bf):
    src = edge_index[0]
    dst = edge_index[1]
    npad = EP - N_EDGES
    dstp = jnp.concatenate(
        [dst, jnp.full((npad,), PAD_DST, jnp.int32)])
    srcp = jnp.concatenate([src, jnp.zeros((npad,), jnp.int32)])
    zeros_tile = jnp.zeros((TILE_WORDS,), jnp.float32)

    partials = _deg_kernel(dstp.reshape(NW, ROWS_A * 128), zeros_tile)
    dinv = _dinv(partials)
    a_flat = _abuild_kernel(dstp.reshape(NS, ROWS_C * 128),
                            srcp.reshape(NS, ROWS_C * 128), zeros_tile)
    A = a_flat.reshape(NP, NP)
    dinv_bc = jnp.broadcast_to(dinv[:, None], (NP, 128))

    xp = jnp.pad(x, ((0, NP - N_NODES), (0, 0)))
    h = xp
    for W, b in ((W1, b1), (W2, b2), (W3, b3), (W4, b4), (W5, b5), (W6, b6)):
        h = _smm(A, _mm(h, W, dinv_bc), b, dinv_bc)
    return _final(h, Wf, bf).reshape(1024)


# speedup vs baseline: 3.3227x; 1.0975x over previous
import functools

import jax
import jax.numpy as jnp
from jax import lax
from jax.experimental import pallas as pl
from jax.experimental.pallas import tpu as pltpu
from jax.experimental.pallas import tpu_sc as plsc

N_NODES = 10000
N_EDGES = 320000
NP = 10240
NC, NS = 2, 16
NW = NC * NS
EP = 327680
PAD_DST = 10100
ROWS_A = 80
ROWS_C = 160
CHUNK_ROWS = 160
N_CHUNKS = NP // CHUNK_ROWS // NC
TILE_ROWS = CHUNK_ROWS // NS
CHUNK_WORDS = CHUNK_ROWS * NP
TILE_WORDS = TILE_ROWS * NP
BR = 8
NB = ROWS_C // (2 * BR)
NEG_SLOPE = 0.01

_mesh = plsc.VectorSubcoreMesh(
    core_axis_name="c", subcore_axis_name="s", num_cores=NC, num_subcores=NS)


@functools.partial(
    pl.kernel,
    out_type=jax.ShapeDtypeStruct((NC, NP), jnp.float32),
    mesh=_mesh,
    scratch_types=[
        pltpu.VMEM((ROWS_A * 128,), jnp.int32),
        pltpu.VMEM((ROWS_A * 128,), jnp.float32),
        pltpu.VMEM_SHARED((NP,), jnp.float32),
    ],
)
def _deg_kernel(dst_hbm, zeros_hbm, out_hbm, idx_v, ones_v, deg_sh):
    c = lax.axis_index("c")
    s = lax.axis_index("s")
    w = s * NC + c
    pltpu.sync_copy(dst_hbm.at[w], idx_v)

    def ones_body(i, _):
        ones_v[pl.ds(i * 16, 16)] = jnp.full((16,), 1.0, jnp.float32)
        return 0

    lax.fori_loop(0, ROWS_A * 8, ones_body, 0)

    @pl.when(s == 0)
    def _():
        pltpu.sync_copy(zeros_hbm.at[pl.ds(0, NP)], deg_sh)

    plsc.subcore_barrier()
    pltpu.sync_copy(ones_v, deg_sh.at[idx_v], add=True)
    plsc.subcore_barrier()

    @pl.when(s == 0)
    def _():
        pltpu.sync_copy(deg_sh, out_hbm.at[c])


def _dinv_body(part_ref, out_ref):
    p = part_ref[...]
    deg = p[0:80, :] + p[80:160, :] + 1.0
    dinv = lax.rsqrt(deg)
    gi = (lax.broadcasted_iota(jnp.int32, (80, 128), 0) * 128
          + lax.broadcasted_iota(jnp.int32, (80, 128), 1))
    out_ref[...] = jnp.where(gi < N_NODES, dinv, 0.0)


def _dinv(partials):
    return pl.pallas_call(
        _dinv_body,
        out_shape=jax.ShapeDtypeStruct((80, 128), jnp.float32),
    )(partials.reshape(160, 128)).reshape(NP)


@functools.partial(
    pl.kernel,
    out_type=jax.ShapeDtypeStruct((NP * NP,), jnp.float32),
    mesh=_mesh,
    scratch_types=[
        pltpu.VMEM((ROWS_C, 128), jnp.int32),
        pltpu.VMEM((BR * 128,), jnp.int32),
        pltpu.VMEM((BR * 128,), jnp.int32),
        pltpu.VMEM((BR * 128,), jnp.float32),
        pltpu.VMEM((BR * 128,), jnp.float32),
        pltpu.VMEM((128,), jnp.int32),
        pltpu.VMEM((128,), jnp.float32),
        pltpu.SemaphoreType.DMA,
        pltpu.SemaphoreType.DMA,
        pltpu.VMEM_SHARED((CHUNK_WORDS,), jnp.float32),
    ],
)
def _abuild_kernel(dst_hbm, src_hbm, zeros_hbm, a_hbm,
                   flat_v, ix0_v, ix1_v, wv0_v, wv1_v, dgi_v, dgv_v,
                   ssem, zsem, chunk_sh):
    c = lax.axis_index("c")
    s = lax.axis_index("s")

    def init_body(t, _):
        pltpu.sync_copy(dst_hbm.at[s, pl.ds(t * BR * 128, BR * 128)], ix0_v)
        pltpu.sync_copy(src_hbm.at[s, pl.ds(t * BR * 128, BR * 128)], ix1_v)
        for r in range(BR):
            for g in range(8):
                sl = pl.ds(g * 16, 16)
                fsl = pl.ds(r * 128 + g * 16, 16)
                flat_v[t * BR + r, sl] = ix0_v[fsl] * NP + ix1_v[fsl]
        return 0

    lax.fori_loop(0, ROWS_C // BR, init_body, 0)

    lanes = lax.iota(jnp.int32, 16)
    my_zero = s * TILE_WORDS

    pltpu.async_copy(zeros_hbm, chunk_sh.at[pl.ds(my_zero, TILE_WORDS)], zsem)

    def fill(j, dst_ix, dst_wv, c0, dump0):
        for r in range(BR):
            row = j * BR + r
            for g in range(8):
                sl = pl.ds(g * 16, 16)
                fsl = pl.ds(r * 128 + g * 16, 16)
                rr = flat_v[row, sl] - c0
                m = (rr >= 0) & (rr < CHUNK_WORDS)
                dump = dump0 + (r * 128 + g * 16) + lanes
                dst_ix[fsl] = jnp.where(m, rr, dump)
                dst_wv[fsl] = jnp.where(m, 1.0, 0.0)

    def chunk_body(cb, _):
        base = (cb * NC + c) * CHUNK_ROWS
        c0 = base * NP

        pltpu.make_async_copy(
            zeros_hbm, chunk_sh.at[pl.ds(my_zero, TILE_WORDS)], zsem).wait()
        plsc.subcore_barrier()

        def sbatch(it, _):
            @pl.when(it >= 1)
            def _():
                pltpu.make_async_copy(
                    wv0_v, chunk_sh.at[ix0_v], ssem).wait()

            fill(it * 2, ix0_v, wv0_v, c0, s * 2048)
            pltpu.async_copy(wv0_v, chunk_sh.at[ix0_v], ssem, add=True)

            @pl.when(it >= 1)
            def _():
                pltpu.make_async_copy(
                    wv1_v, chunk_sh.at[ix1_v], ssem).wait()

            fill(it * 2 + 1, ix1_v, wv1_v, c0, s * 2048 + 1024)
            pltpu.async_copy(wv1_v, chunk_sh.at[ix1_v], ssem, add=True)
            return 0

        lax.fori_loop(0, NB, sbatch, 0)

        for g in range(1, 8):
            sl = pl.ds(g * 16, 16)
            dgi_v[sl] = jnp.zeros((16,), jnp.int32)
            dgv_v[sl] = jnp.zeros((16,), jnp.float32)
        l = s * TILE_ROWS + lanes
        dm = lanes < TILE_ROWS
        dgi_v[pl.ds(0, 16)] = jnp.where(dm, l * NP + base + l, 0)
        dgv_v[pl.ds(0, 16)] = jnp.where(dm, 1.0, 0.0)
        pltpu.async_copy(dgv_v, chunk_sh.at[dgi_v], ssem, add=True)

        pltpu.make_async_copy(wv0_v, chunk_sh.at[ix0_v], ssem).wait()
        pltpu.make_async_copy(wv1_v, chunk_sh.at[ix1_v], ssem).wait()
        pltpu.make_async_copy(dgv_v, chunk_sh.at[dgi_v], ssem).wait()
        plsc.subcore_barrier()

        pltpu.sync_copy(
            chunk_sh.at[pl.ds(my_zero, TILE_WORDS)],
            a_hbm.at[pl.ds(base * NP + my_zero, TILE_WORDS)])
        pltpu.async_copy(
            zeros_hbm, chunk_sh.at[pl.ds(my_zero, TILE_WORDS)], zsem)
        return 0

    lax.fori_loop(0, N_CHUNKS, chunk_body, 0)
    pltpu.make_async_copy(
        zeros_hbm, chunk_sh.at[pl.ds(my_zero, TILE_WORDS)], zsem).wait()


def _cast_body(a_ref, o_ref):
    o_ref[...] = a_ref[...].astype(jnp.bfloat16)


def _cast_bf16(A):
    return pl.pallas_call(
        _cast_body,
        grid=(NP // 256,),
        in_specs=[pl.BlockSpec((256, NP), lambda i: (i, 0))],
        out_specs=pl.BlockSpec((256, NP), lambda i: (i, 0)),
        out_shape=jax.ShapeDtypeStruct((NP, NP), jnp.bfloat16),
    )(A)


def _mm_body(h_ref, w_ref, d_ref, o_ref):
    o_ref[...] = (jnp.dot(h_ref[...], w_ref[...],
                          preferred_element_type=jnp.float32)
                  * d_ref[:, 0:1]).astype(jnp.bfloat16)


def _mm(h, w, dinv_bc):
    m, din = h.shape
    dout = w.shape[1]
    return pl.pallas_call(
        _mm_body,
        grid=(m // 256,),
        in_specs=[
            pl.BlockSpec((256, din), lambda i: (i, 0)),
            pl.BlockSpec((din, dout), lambda i: (0, 0)),
            pl.BlockSpec((256, 128), lambda i: (i, 0)),
        ],
        out_specs=pl.BlockSpec((256, dout), lambda i: (i, 0)),
        out_shape=jax.ShapeDtypeStruct((m, dout), jnp.bfloat16),
    )(h, w, dinv_bc)


def _smm_body(nk, s_ref, h_ref, b_ref, d_ref, o_ref, acc_ref):
    k = pl.program_id(1)

    @pl.when(k == 0)
    def _():
        acc_ref[...] = jnp.zeros_like(acc_ref)

    acc_ref[...] += jnp.dot(s_ref[...], h_ref[...],
                            preferred_element_type=jnp.float32)

    @pl.when(k == nk - 1)
    def _():
        t = acc_ref[...] * d_ref[:, 0:1] + b_ref[...]
        o_ref[...] = jnp.where(t > 0, t, NEG_SLOPE * t)


def _smm(S, h, b, dinv_bc):
    dout = h.shape[1]
    nk = NP // 512
    return pl.pallas_call(
        functools.partial(_smm_body, nk),
        grid=(NP // 256, nk),
        in_specs=[
            pl.BlockSpec((256, 512), lambda i, k: (i, k)),
            pl.BlockSpec((512, dout), lambda i, k: (k, 0)),
            pl.BlockSpec((1, dout), lambda i, k: (0, 0)),
            pl.BlockSpec((256, 128), lambda i, k: (i, 0)),
        ],
        out_specs=pl.BlockSpec((256, dout), lambda i, k: (i, 0)),
        out_shape=jax.ShapeDtypeStruct((NP, dout), jnp.float32),
        scratch_shapes=[pltpu.VMEM((256, dout), jnp.float32)],
        compiler_params=pltpu.CompilerParams(
            dimension_semantics=("parallel", "arbitrary")),
    )(S, h, b.reshape(1, dout), dinv_bc)


def _final_body(nm, h_ref, wf_ref, bf_ref, o_ref, acc_ref):
    i = pl.program_id(0)

    @pl.when(i == 0)
    def _():
        acc_ref[...] = jnp.zeros_like(acc_ref)

    row = i * 256 + lax.broadcasted_iota(jnp.int32, (256, 256), 0)
    x = jnp.where(row < N_NODES, h_ref[...], 0.0)
    acc_ref[...] += jnp.sum(x, axis=0, keepdims=True)

    @pl.when(i == nm - 1)
    def _():
        o_ref[...] = (jnp.dot(acc_ref[...], wf_ref[...],
                              preferred_element_type=jnp.float32)
                      + float(N_NODES) * bf_ref[...])


def _final(h, wf, bf):
    nm = NP // 256
    return pl.pallas_call(
        functools.partial(_final_body, nm),
        grid=(nm,),
        in_specs=[
            pl.BlockSpec((256, 256), lambda i: (i, 0)),
            pl.BlockSpec((256, 1024), lambda i: (0, 0)),
            pl.BlockSpec((1, 1024), lambda i: (0, 0)),
        ],
        out_specs=pl.BlockSpec((1, 1024), lambda i: (0, 0)),
        out_shape=jax.ShapeDtypeStruct((1, 1024), jnp.float32),
        scratch_shapes=[pltpu.VMEM((1, 256), jnp.float32)],
    )(h, wf, bf.reshape(1, 1024))


def kernel(x, edge_index, W1, b1, W2, b2, W3, b3, W4, b4, W5, b5, W6, b6,
           Wf, bf):
    src = edge_index[0]
    dst = edge_index[1]
    npad = EP - N_EDGES
    dstp = jnp.concatenate(
        [dst, jnp.full((npad,), PAD_DST, jnp.int32)])
    srcp = jnp.concatenate([src, jnp.zeros((npad,), jnp.int32)])
    zeros_tile = jnp.zeros((TILE_WORDS,), jnp.float32)

    partials = _deg_kernel(dstp.reshape(NW, ROWS_A * 128), zeros_tile)
    dinv = _dinv(partials)
    a_flat = _abuild_kernel(dstp.reshape(NS, ROWS_C * 128),
                            srcp.reshape(NS, ROWS_C * 128), zeros_tile)
    A = a_flat.reshape(NP, NP)
    dinv_bc = jnp.broadcast_to(dinv[:, None], (NP, 128))

    xp = jnp.pad(x, ((0, NP - N_NODES), (0, 0)))
    h = xp
    Ab = _cast_bf16(A)
    for W, b in ((W1, b1), (W2, b2), (W3, b3), (W4, b4), (W5, b5), (W6, b6)):
        h = _smm(Ab, _mm(h, W, dinv_bc), b, dinv_bc)
    return _final(h, Wf, bf).reshape(1024)


# speedup vs baseline: 7.5440x; 2.2705x over previous
import functools

import jax
import jax.numpy as jnp
from jax import lax
from jax.experimental import pallas as pl
from jax.experimental.pallas import tpu as pltpu
from jax.experimental.pallas import tpu_sc as plsc

N_NODES = 10000
N_EDGES = 320000
NP = 10240
NC, NS = 2, 16
NW = NC * NS
EP = 327680
PAD_DST = 10100
ROWS_A = 80
ROWS_C = 160
CHUNK_ROWS = 160
N_CHUNKS = NP // CHUNK_ROWS // NC
TILE_ROWS = CHUNK_ROWS // NS
CHUNK_WORDS = CHUNK_ROWS * NP
TILE_WORDS = TILE_ROWS * NP
BR = 8
NB = ROWS_C // (2 * BR)
NEG_SLOPE = 0.01

_mesh = plsc.VectorSubcoreMesh(
    core_axis_name="c", subcore_axis_name="s", num_cores=NC, num_subcores=NS)


@functools.partial(
    pl.kernel,
    out_type=jax.ShapeDtypeStruct((NC, NP), jnp.float32),
    mesh=_mesh,
    scratch_types=[
        pltpu.VMEM((ROWS_A * 128,), jnp.int32),
        pltpu.VMEM((ROWS_A * 128,), jnp.float32),
        pltpu.VMEM_SHARED((NP,), jnp.float32),
    ],
)
def _deg_kernel(dst_hbm, zeros_hbm, out_hbm, idx_v, ones_v, deg_sh):
    c = lax.axis_index("c")
    s = lax.axis_index("s")
    w = s * NC + c
    pltpu.sync_copy(dst_hbm.at[w], idx_v)

    def ones_body(i, _):
        ones_v[pl.ds(i * 16, 16)] = jnp.full((16,), 1.0, jnp.float32)
        return 0

    lax.fori_loop(0, ROWS_A * 8, ones_body, 0)

    @pl.when(s == 0)
    def _():
        pltpu.sync_copy(zeros_hbm.at[pl.ds(0, NP)], deg_sh)

    plsc.subcore_barrier()
    pltpu.sync_copy(ones_v, deg_sh.at[idx_v], add=True)
    plsc.subcore_barrier()

    @pl.when(s == 0)
    def _():
        pltpu.sync_copy(deg_sh, out_hbm.at[c])


def _dinv_body(part_ref, out_ref):
    p = part_ref[...]
    deg = p[0:80, :] + p[80:160, :] + 1.0
    dinv = lax.rsqrt(deg)
    gi = (lax.broadcasted_iota(jnp.int32, (80, 128), 0) * 128
          + lax.broadcasted_iota(jnp.int32, (80, 128), 1))
    out_ref[...] = jnp.where(gi < N_NODES, dinv, 0.0)


def _dinv(partials):
    return pl.pallas_call(
        _dinv_body,
        out_shape=jax.ShapeDtypeStruct((80, 128), jnp.float32),
    )(partials.reshape(160, 128)).reshape(NP)


@functools.partial(
    pl.kernel,
    out_type=jax.ShapeDtypeStruct((NP * NP,), jnp.float32),
    mesh=_mesh,
    scratch_types=[
        pltpu.VMEM((ROWS_C, 128), jnp.int32),
        pltpu.VMEM((BR * 128,), jnp.int32),
        pltpu.VMEM((BR * 128,), jnp.int32),
        pltpu.VMEM((BR * 128,), jnp.float32),
        pltpu.VMEM((BR * 128,), jnp.float32),
        pltpu.VMEM((128,), jnp.int32),
        pltpu.VMEM((128,), jnp.float32),
        pltpu.SemaphoreType.DMA,
        pltpu.SemaphoreType.DMA,
        pltpu.VMEM_SHARED((CHUNK_WORDS,), jnp.float32),
    ],
)
def _abuild_kernel(dst_hbm, src_hbm, zeros_hbm, a_hbm,
                   flat_v, ix0_v, ix1_v, wv0_v, wv1_v, dgi_v, dgv_v,
                   ssem, zsem, chunk_sh):
    c = lax.axis_index("c")
    s = lax.axis_index("s")

    def init_body(t, _):
        pltpu.sync_copy(dst_hbm.at[s, pl.ds(t * BR * 128, BR * 128)], ix0_v)
        pltpu.sync_copy(src_hbm.at[s, pl.ds(t * BR * 128, BR * 128)], ix1_v)
        for r in range(BR):
            for g in range(8):
                sl = pl.ds(g * 16, 16)
                fsl = pl.ds(r * 128 + g * 16, 16)
                flat_v[t * BR + r, sl] = ix0_v[fsl] * NP + ix1_v[fsl]
        return 0

    lax.fori_loop(0, ROWS_C // BR, init_body, 0)

    lanes = lax.iota(jnp.int32, 16)
    my_zero = s * TILE_WORDS

    pltpu.async_copy(zeros_hbm, chunk_sh.at[pl.ds(my_zero, TILE_WORDS)], zsem)

    def fill(j, dst_ix, dst_wv, c0, dump0):
        for r in range(BR):
            row = j * BR + r
            for g in range(8):
                sl = pl.ds(g * 16, 16)
                fsl = pl.ds(r * 128 + g * 16, 16)
                rr = flat_v[row, sl] - c0
                m = (rr >= 0) & (rr < CHUNK_WORDS)
                dump = dump0 + (r * 128 + g * 16) + lanes
                dst_ix[fsl] = jnp.where(m, rr, dump)
                dst_wv[fsl] = jnp.where(m, 1.0, 0.0)

    def chunk_body(cb, _):
        base = (cb * NC + c) * CHUNK_ROWS
        c0 = base * NP

        pltpu.make_async_copy(
            zeros_hbm, chunk_sh.at[pl.ds(my_zero, TILE_WORDS)], zsem).wait()
        plsc.subcore_barrier()

        def sbatch(it, _):
            @pl.when(it >= 1)
            def _():
                pltpu.make_async_copy(
                    wv0_v, chunk_sh.at[ix0_v], ssem).wait()

            fill(it * 2, ix0_v, wv0_v, c0, s * 2048)
            pltpu.async_copy(wv0_v, chunk_sh.at[ix0_v], ssem, add=True)

            @pl.when(it >= 1)
            def _():
                pltpu.make_async_copy(
                    wv1_v, chunk_sh.at[ix1_v], ssem).wait()

            fill(it * 2 + 1, ix1_v, wv1_v, c0, s * 2048 + 1024)
            pltpu.async_copy(wv1_v, chunk_sh.at[ix1_v], ssem, add=True)
            return 0

        lax.fori_loop(0, NB, sbatch, 0)

        for g in range(1, 8):
            sl = pl.ds(g * 16, 16)
            dgi_v[sl] = jnp.zeros((16,), jnp.int32)
            dgv_v[sl] = jnp.zeros((16,), jnp.float32)
        l = s * TILE_ROWS + lanes
        dm = lanes < TILE_ROWS
        dgi_v[pl.ds(0, 16)] = jnp.where(dm, l * NP + base + l, 0)
        dgv_v[pl.ds(0, 16)] = jnp.where(dm, 1.0, 0.0)
        pltpu.async_copy(dgv_v, chunk_sh.at[dgi_v], ssem, add=True)

        pltpu.make_async_copy(wv0_v, chunk_sh.at[ix0_v], ssem).wait()
        pltpu.make_async_copy(wv1_v, chunk_sh.at[ix1_v], ssem).wait()
        pltpu.make_async_copy(dgv_v, chunk_sh.at[dgi_v], ssem).wait()
        plsc.subcore_barrier()

        pltpu.sync_copy(
            chunk_sh.at[pl.ds(my_zero, TILE_WORDS)],
            a_hbm.at[pl.ds(base * NP + my_zero, TILE_WORDS)])
        pltpu.async_copy(
            zeros_hbm, chunk_sh.at[pl.ds(my_zero, TILE_WORDS)], zsem)
        return 0

    lax.fori_loop(0, N_CHUNKS, chunk_body, 0)
    pltpu.make_async_copy(
        zeros_hbm, chunk_sh.at[pl.ds(my_zero, TILE_WORDS)], zsem).wait()


def _cast_body(a_ref, o_ref):
    o_ref[...] = a_ref[...].astype(jnp.bfloat16)


def _cast_bf16(A):
    return pl.pallas_call(
        _cast_body,
        grid=(NP // 256,),
        in_specs=[pl.BlockSpec((256, NP), lambda i: (i, 0))],
        out_specs=pl.BlockSpec((256, NP), lambda i: (i, 0)),
        out_shape=jax.ShapeDtypeStruct((NP, NP), jnp.bfloat16),
    )(A)


def _mm_body(h_ref, w_ref, d_ref, o_ref):
    o_ref[...] = (jnp.dot(h_ref[...], w_ref[...],
                          preferred_element_type=jnp.float32)
                  * d_ref[:, 0:1]).astype(jnp.bfloat16)


def _mm(h, w, dinv_bc):
    m, din = h.shape
    dout = w.shape[1]
    return pl.pallas_call(
        _mm_body,
        grid=(m // 256,),
        in_specs=[
            pl.BlockSpec((256, din), lambda i: (i, 0)),
            pl.BlockSpec((din, dout), lambda i: (0, 0)),
            pl.BlockSpec((256, 128), lambda i: (i, 0)),
        ],
        out_specs=pl.BlockSpec((256, dout), lambda i: (i, 0)),
        out_shape=jax.ShapeDtypeStruct((m, dout), jnp.bfloat16),
    )(h, w, dinv_bc)


def _smm_body(s_ref, h_ref, b_ref, d_ref, o_ref):
    t = (jnp.dot(s_ref[...], h_ref[...], preferred_element_type=jnp.float32)
         * d_ref[:, 0:1] + b_ref[...])
    o_ref[...] = jnp.where(t > 0, t, NEG_SLOPE * t)


def _smm(S, h, b, dinv_bc):
    dout = h.shape[1]
    return pl.pallas_call(
        _smm_body,
        grid=(NP // 256,),
        in_specs=[
            pl.BlockSpec((256, NP), lambda i: (i, 0)),
            pl.BlockSpec((NP, dout), lambda i: (0, 0)),
            pl.BlockSpec((1, dout), lambda i: (0, 0)),
            pl.BlockSpec((256, 128), lambda i: (i, 0)),
        ],
        out_specs=pl.BlockSpec((256, dout), lambda i: (i, 0)),
        out_shape=jax.ShapeDtypeStruct((NP, dout), jnp.float32),
    )(S, h, b.reshape(1, dout), dinv_bc)


def _final_body(nm, h_ref, wf_ref, bf_ref, o_ref, acc_ref):
    i = pl.program_id(0)

    @pl.when(i == 0)
    def _():
        acc_ref[...] = jnp.zeros_like(acc_ref)

    row = i * 256 + lax.broadcasted_iota(jnp.int32, (256, 256), 0)
    x = jnp.where(row < N_NODES, h_ref[...], 0.0)
    acc_ref[...] += jnp.sum(x, axis=0, keepdims=True)

    @pl.when(i == nm - 1)
    def _():
        o_ref[...] = (jnp.dot(acc_ref[...], wf_ref[...],
                              preferred_element_type=jnp.float32)
                      + float(N_NODES) * bf_ref[...])


def _final(h, wf, bf):
    nm = NP // 256
    return pl.pallas_call(
        functools.partial(_final_body, nm),
        grid=(nm,),
        in_specs=[
            pl.BlockSpec((256, 256), lambda i: (i, 0)),
            pl.BlockSpec((256, 1024), lambda i: (0, 0)),
            pl.BlockSpec((1, 1024), lambda i: (0, 0)),
        ],
        out_specs=pl.BlockSpec((1, 1024), lambda i: (0, 0)),
        out_shape=jax.ShapeDtypeStruct((1, 1024), jnp.float32),
        scratch_shapes=[pltpu.VMEM((1, 256), jnp.float32)],
    )(h, wf, bf.reshape(1, 1024))


def kernel(x, edge_index, W1, b1, W2, b2, W3, b3, W4, b4, W5, b5, W6, b6,
           Wf, bf):
    src = edge_index[0]
    dst = edge_index[1]
    npad = EP - N_EDGES
    dstp = jnp.concatenate(
        [dst, jnp.full((npad,), PAD_DST, jnp.int32)])
    srcp = jnp.concatenate([src, jnp.zeros((npad,), jnp.int32)])
    zeros_tile = jnp.zeros((TILE_WORDS,), jnp.float32)

    partials = _deg_kernel(dstp.reshape(NW, ROWS_A * 128), zeros_tile)
    dinv = _dinv(partials)
    a_flat = _abuild_kernel(dstp.reshape(NS, ROWS_C * 128),
                            srcp.reshape(NS, ROWS_C * 128), zeros_tile)
    A = a_flat.reshape(NP, NP)
    dinv_bc = jnp.broadcast_to(dinv[:, None], (NP, 128))

    xp = jnp.pad(x, ((0, NP - N_NODES), (0, 0)))
    h = xp
    Ab = _cast_bf16(A)
    for W, b in ((W1, b1), (W2, b2), (W3, b3), (W4, b4), (W5, b5), (W6, b6)):
        h = _smm(Ab, _mm(h, W, dinv_bc), b, dinv_bc)
    return _final(h, Wf, bf).reshape(1024)


# speedup vs baseline: 8.0844x; 1.0716x over previous
import functools

import jax
import jax.numpy as jnp
from jax import lax
from jax.experimental import pallas as pl
from jax.experimental.pallas import tpu as pltpu
from jax.experimental.pallas import tpu_sc as plsc

N_NODES = 10000
N_EDGES = 320000
NP = 10240
NC, NS = 2, 16
NW = NC * NS
EP = 327680
PAD_DST = 10100
ROWS_A = 80
ROWS_C = 160
CHUNK_ROWS = 160
N_CHUNKS = NP // CHUNK_ROWS // NC
TILE_ROWS = CHUNK_ROWS // NS
CHUNK_WORDS = CHUNK_ROWS * NP
TILE_WORDS = TILE_ROWS * NP
BR = 8
NB = ROWS_C // (2 * BR)
NEG_SLOPE = 0.01

_mesh = plsc.VectorSubcoreMesh(
    core_axis_name="c", subcore_axis_name="s", num_cores=NC, num_subcores=NS)


@functools.partial(
    pl.kernel,
    out_type=jax.ShapeDtypeStruct((NC, NP), jnp.float32),
    mesh=_mesh,
    scratch_types=[
        pltpu.VMEM((ROWS_A * 128,), jnp.int32),
        pltpu.VMEM((ROWS_A * 128,), jnp.float32),
        pltpu.VMEM_SHARED((NP,), jnp.float32),
    ],
)
def _deg_kernel(dst_hbm, zeros_hbm, out_hbm, idx_v, ones_v, deg_sh):
    c = lax.axis_index("c")
    s = lax.axis_index("s")
    w = s * NC + c
    pltpu.sync_copy(dst_hbm.at[w], idx_v)

    def ones_body(i, _):
        ones_v[pl.ds(i * 16, 16)] = jnp.full((16,), 1.0, jnp.float32)
        return 0

    lax.fori_loop(0, ROWS_A * 8, ones_body, 0)

    @pl.when(s == 0)
    def _():
        pltpu.sync_copy(zeros_hbm.at[pl.ds(0, NP)], deg_sh)

    plsc.subcore_barrier()
    pltpu.sync_copy(ones_v, deg_sh.at[idx_v], add=True)
    plsc.subcore_barrier()

    @pl.when(s == 0)
    def _():
        pltpu.sync_copy(deg_sh, out_hbm.at[c])


def _dinv_body(part_ref, out_ref):
    p = part_ref[...]
    deg = p[0:80, :] + p[80:160, :] + 1.0
    dinv = lax.rsqrt(deg)
    gi = (lax.broadcasted_iota(jnp.int32, (80, 128), 0) * 128
          + lax.broadcasted_iota(jnp.int32, (80, 128), 1))
    out_ref[...] = jnp.where(gi < N_NODES, dinv, 0.0)


def _dinv(partials):
    return pl.pallas_call(
        _dinv_body,
        out_shape=jax.ShapeDtypeStruct((80, 128), jnp.float32),
    )(partials.reshape(160, 128)).reshape(NP)


@functools.partial(
    pl.kernel,
    out_type=jax.ShapeDtypeStruct((NP * NP,), jnp.float32),
    mesh=_mesh,
    scratch_types=[
        pltpu.VMEM((ROWS_C, 128), jnp.int32),
        pltpu.VMEM((BR * 128,), jnp.int32),
        pltpu.VMEM((BR * 128,), jnp.int32),
        pltpu.VMEM((BR * 128,), jnp.float32),
        pltpu.VMEM((BR * 128,), jnp.float32),
        pltpu.VMEM((128,), jnp.int32),
        pltpu.VMEM((128,), jnp.float32),
        pltpu.SemaphoreType.DMA,
        pltpu.SemaphoreType.DMA,
        pltpu.VMEM_SHARED((CHUNK_WORDS,), jnp.float32),
    ],
)
def _abuild_kernel(dst_hbm, src_hbm, zeros_hbm, a_hbm,
                   flat_v, ix0_v, ix1_v, wv0_v, wv1_v, dgi_v, dgv_v,
                   ssem, zsem, chunk_sh):
    c = lax.axis_index("c")
    s = lax.axis_index("s")

    def init_body(t, _):
        pltpu.sync_copy(dst_hbm.at[s, pl.ds(t * BR * 128, BR * 128)], ix0_v)
        pltpu.sync_copy(src_hbm.at[s, pl.ds(t * BR * 128, BR * 128)], ix1_v)
        for r in range(BR):
            for g in range(8):
                sl = pl.ds(g * 16, 16)
                fsl = pl.ds(r * 128 + g * 16, 16)
                flat_v[t * BR + r, sl] = ix0_v[fsl] * NP + ix1_v[fsl]
        return 0

    lax.fori_loop(0, ROWS_C // BR, init_body, 0)

    lanes = lax.iota(jnp.int32, 16)
    my_zero = s * TILE_WORDS

    pltpu.async_copy(zeros_hbm, chunk_sh.at[pl.ds(my_zero, TILE_WORDS)], zsem)

    def fill(j, dst_ix, dst_wv, c0, dump0):
        for r in range(BR):
            row = j * BR + r
            for g in range(8):
                sl = pl.ds(g * 16, 16)
                fsl = pl.ds(r * 128 + g * 16, 16)
                rr = flat_v[row, sl] - c0
                m = (rr >= 0) & (rr < CHUNK_WORDS)
                dump = dump0 + (r * 128 + g * 16) + lanes
                dst_ix[fsl] = jnp.where(m, rr, dump)
                dst_wv[fsl] = jnp.where(m, 1.0, 0.0)

    def chunk_body(cb, _):
        base = (cb * NC + c) * CHUNK_ROWS
        c0 = base * NP

        pltpu.make_async_copy(
            zeros_hbm, chunk_sh.at[pl.ds(my_zero, TILE_WORDS)], zsem).wait()
        plsc.subcore_barrier()

        def sbatch(it, _):
            @pl.when(it >= 1)
            def _():
                pltpu.make_async_copy(
                    wv0_v, chunk_sh.at[ix0_v], ssem).wait()

            fill(it * 2, ix0_v, wv0_v, c0, s * 2048)
            pltpu.async_copy(wv0_v, chunk_sh.at[ix0_v], ssem, add=True)

            @pl.when(it >= 1)
            def _():
                pltpu.make_async_copy(
                    wv1_v, chunk_sh.at[ix1_v], ssem).wait()

            fill(it * 2 + 1, ix1_v, wv1_v, c0, s * 2048 + 1024)
            pltpu.async_copy(wv1_v, chunk_sh.at[ix1_v], ssem, add=True)
            return 0

        lax.fori_loop(0, NB, sbatch, 0)

        for g in range(1, 8):
            sl = pl.ds(g * 16, 16)
            dgi_v[sl] = jnp.zeros((16,), jnp.int32)
            dgv_v[sl] = jnp.zeros((16,), jnp.float32)
        l = s * TILE_ROWS + lanes
        dm = lanes < TILE_ROWS
        dgi_v[pl.ds(0, 16)] = jnp.where(dm, l * NP + base + l, 0)
        dgv_v[pl.ds(0, 16)] = jnp.where(dm, 1.0, 0.0)
        pltpu.async_copy(dgv_v, chunk_sh.at[dgi_v], ssem, add=True)

        pltpu.make_async_copy(wv0_v, chunk_sh.at[ix0_v], ssem).wait()
        pltpu.make_async_copy(wv1_v, chunk_sh.at[ix1_v], ssem).wait()
        pltpu.make_async_copy(dgv_v, chunk_sh.at[dgi_v], ssem).wait()
        plsc.subcore_barrier()

        pltpu.sync_copy(
            chunk_sh.at[pl.ds(my_zero, TILE_WORDS)],
            a_hbm.at[pl.ds(base * NP + my_zero, TILE_WORDS)])
        pltpu.async_copy(
            zeros_hbm, chunk_sh.at[pl.ds(my_zero, TILE_WORDS)], zsem)
        return 0

    lax.fori_loop(0, N_CHUNKS, chunk_body, 0)
    pltpu.make_async_copy(
        zeros_hbm, chunk_sh.at[pl.ds(my_zero, TILE_WORDS)], zsem).wait()


def _cast_body(a_ref, o_ref):
    o_ref[...] = a_ref[...].astype(jnp.bfloat16)


def _cast_bf16(A):
    return pl.pallas_call(
        _cast_body,
        grid=(NP // 256,),
        in_specs=[pl.BlockSpec((256, NP), lambda i: (i, 0))],
        out_specs=pl.BlockSpec((256, NP), lambda i: (i, 0)),
        out_shape=jax.ShapeDtypeStruct((NP, NP), jnp.bfloat16),
    )(A)


def _mm_body(h_ref, w_ref, d_ref, o_ref):
    o_ref[...] = (jnp.dot(h_ref[...], w_ref[...],
                          preferred_element_type=jnp.float32)
                  * d_ref[:, 0:1]).astype(jnp.bfloat16)


def _mm(h, w, dinv_bc):
    m, din = h.shape
    dout = w.shape[1]
    return pl.pallas_call(
        _mm_body,
        grid=(m // 256,),
        in_specs=[
            pl.BlockSpec((256, din), lambda i: (i, 0)),
            pl.BlockSpec((din, dout), lambda i: (0, 0)),
            pl.BlockSpec((256, 128), lambda i: (i, 0)),
        ],
        out_specs=pl.BlockSpec((256, dout), lambda i: (i, 0)),
        out_shape=jax.ShapeDtypeStruct((m, dout), jnp.bfloat16),
    )(h, w, dinv_bc)


def _smm_body(s_ref, h_ref, b_ref, d_ref, o_ref):
    t = (jnp.dot(s_ref[...], h_ref[...], preferred_element_type=jnp.float32)
         * d_ref[:, 0:1] + b_ref[...])
    o_ref[...] = jnp.where(t > 0, t, NEG_SLOPE * t)


def _smmf_body(s_ref, h_ref, b_ref, d_ref, w_ref, dn_ref, o_ref):
    t = (jnp.dot(s_ref[...], h_ref[...], preferred_element_type=jnp.float32)
         * d_ref[:, 0:1] + b_ref[...])
    t = jnp.where(t > 0, t, NEG_SLOPE * t)
    o_ref[...] = (jnp.dot(t, w_ref[...], preferred_element_type=jnp.float32)
                  * dn_ref[:, 0:1]).astype(jnp.bfloat16)


def _smm_fused(S, h, b, dinv_bc, w_next):
    dout = h.shape[1]
    dout2 = w_next.shape[1]
    return pl.pallas_call(
        _smmf_body,
        grid=(NP // 256,),
        in_specs=[
            pl.BlockSpec((256, NP), lambda i: (i, 0)),
            pl.BlockSpec((NP, dout), lambda i: (0, 0)),
            pl.BlockSpec((1, dout), lambda i: (0, 0)),
            pl.BlockSpec((256, 128), lambda i: (i, 0)),
            pl.BlockSpec((dout, dout2), lambda i: (0, 0)),
            pl.BlockSpec((256, 128), lambda i: (i, 0)),
        ],
        out_specs=pl.BlockSpec((256, dout2), lambda i: (i, 0)),
        out_shape=jax.ShapeDtypeStruct((NP, dout2), jnp.bfloat16),
    )(S, h, b.reshape(1, dout), dinv_bc, w_next, dinv_bc)


def _smm(S, h, b, dinv_bc):
    dout = h.shape[1]
    return pl.pallas_call(
        _smm_body,
        grid=(NP // 256,),
        in_specs=[
            pl.BlockSpec((256, NP), lambda i: (i, 0)),
            pl.BlockSpec((NP, dout), lambda i: (0, 0)),
            pl.BlockSpec((1, dout), lambda i: (0, 0)),
            pl.BlockSpec((256, 128), lambda i: (i, 0)),
        ],
        out_specs=pl.BlockSpec((256, dout), lambda i: (i, 0)),
        out_shape=jax.ShapeDtypeStruct((NP, dout), jnp.float32),
    )(S, h, b.reshape(1, dout), dinv_bc)


def _final_body(nm, h_ref, wf_ref, bf_ref, o_ref, acc_ref):
    i = pl.program_id(0)

    @pl.when(i == 0)
    def _():
        acc_ref[...] = jnp.zeros_like(acc_ref)

    row = i * 256 + lax.broadcasted_iota(jnp.int32, (256, 256), 0)
    x = jnp.where(row < N_NODES, h_ref[...], 0.0)
    acc_ref[...] += jnp.sum(x, axis=0, keepdims=True)

    @pl.when(i == nm - 1)
    def _():
        o_ref[...] = (jnp.dot(acc_ref[...], wf_ref[...],
                              preferred_element_type=jnp.float32)
                      + float(N_NODES) * bf_ref[...])


def _final(h, wf, bf):
    nm = NP // 256
    return pl.pallas_call(
        functools.partial(_final_body, nm),
        grid=(nm,),
        in_specs=[
            pl.BlockSpec((256, 256), lambda i: (i, 0)),
            pl.BlockSpec((256, 1024), lambda i: (0, 0)),
            pl.BlockSpec((1, 1024), lambda i: (0, 0)),
        ],
        out_specs=pl.BlockSpec((1, 1024), lambda i: (0, 0)),
        out_shape=jax.ShapeDtypeStruct((1, 1024), jnp.float32),
        scratch_shapes=[pltpu.VMEM((1, 256), jnp.float32)],
    )(h, wf, bf.reshape(1, 1024))


def kernel(x, edge_index, W1, b1, W2, b2, W3, b3, W4, b4, W5, b5, W6, b6,
           Wf, bf):
    src = edge_index[0]
    dst = edge_index[1]
    npad = EP - N_EDGES
    dstp = jnp.concatenate(
        [dst, jnp.full((npad,), PAD_DST, jnp.int32)])
    srcp = jnp.concatenate([src, jnp.zeros((npad,), jnp.int32)])
    zeros_tile = jnp.zeros((TILE_WORDS,), jnp.float32)

    partials = _deg_kernel(dstp.reshape(NW, ROWS_A * 128), zeros_tile)
    dinv = _dinv(partials)
    a_flat = _abuild_kernel(dstp.reshape(NS, ROWS_C * 128),
                            srcp.reshape(NS, ROWS_C * 128), zeros_tile)
    A = a_flat.reshape(NP, NP)
    dinv_bc = jnp.broadcast_to(dinv[:, None], (NP, 128))

    xp = jnp.pad(x, ((0, NP - N_NODES), (0, 0)))
    Ab = _cast_bf16(A)
    ws = (W2, W3, W4, W5, W6)
    bs = (b1, b2, b3, b4, b5)
    ht = _mm(xp, W1, dinv_bc)
    for W, b in zip(ws, bs):
        ht = _smm_fused(Ab, ht, b, dinv_bc, W)
    h = _smm(Ab, ht, b6, dinv_bc)
    return _final(h, Wf, bf).reshape(1024)
